# baseline (speedup 1.0000x reference)
import functools

import jax
import jax.numpy as jnp
from jax.experimental import pallas as pl
from jax.experimental.pallas import tpu as pltpu

N = 10000
D = 128
H2 = 128
CLUSTERS = 64
LATENT = 64
BLK = 2000

_R_WD1 = 0
_R_WD2 = 128
_R_WMU = 160
_R_WLV = 288
_R_WP1 = 416
_R_WP2 = 480
_R_VEC = 512
_R_TOT = 528


def _dense_kernel(nodes_ref, w1_ref, w2_ref, pk_ref,
                  assign_ref, coarse_ref, mu_ref, lv_ref, py_ref):
    i = pl.program_id(0)
    b1 = pk_ref[_R_VEC, :].reshape(1, D)
    b2 = pk_ref[_R_VEC + 1, :].reshape(1, D)
    x = nodes_ref[...]
    h = jnp.dot(x, w1_ref[...], preferred_element_type=jnp.float32) + b1
    h = jnp.maximum(h, 0.0)
    h = jnp.dot(h, w2_ref[...], preferred_element_type=jnp.float32) + b2

    wd1 = pk_ref[_R_WD1:_R_WD1 + 128, :32]
    wd2 = pk_ref[_R_WD2:_R_WD2 + 32, :64]
    bd1 = pk_ref[_R_VEC + 2, :32].reshape(1, 32)
    bd2 = pk_ref[_R_VEC + 3, :64].reshape(1, 64)
    a = jnp.dot(h, wd1, preferred_element_type=jnp.float32) + bd1
    a = jnp.maximum(a, 0.0)
    logits = jnp.dot(a, wd2, preferred_element_type=jnp.float32) + bd2
    m = jnp.max(logits, axis=-1, keepdims=True)
    e = jnp.exp(logits - m)
    assign = e / jnp.sum(e, axis=-1, keepdims=True)
    assign_ref[...] = assign

    partial = jax.lax.dot_general(assign, h, (((0,), (0,)), ((), ())),
                                  preferred_element_type=jnp.float32)

    @pl.when(i == 0)
    def _():
        coarse_ref[...] = partial

    @pl.when(i > 0)
    def _():
        coarse_ref[...] += partial

    @pl.when(i == pl.num_programs(0) - 1)
    def _():
        coarse = coarse_ref[...]
        macro = jnp.mean(coarse, axis=0, keepdims=True)
        wmu = pk_ref[_R_WMU:_R_WMU + 128, :64]
        wlv = pk_ref[_R_WLV:_R_WLV + 128, :64]
        wp1 = pk_ref[_R_WP1:_R_WP1 + 64, :32]
        wp2 = pk_ref[_R_WP2:_R_WP2 + 32, :1]
        bmu = pk_ref[_R_VEC + 4, :64].reshape(1, 64)
        blv = pk_ref[_R_VEC + 5, :64].reshape(1, 64)
        bp1 = pk_ref[_R_VEC + 6, :32].reshape(1, 32)
        bp2 = pk_ref[_R_VEC + 7, :1].reshape(1, 1)
        eps = pk_ref[_R_VEC + 8, :64].reshape(1, 64)
        mu = jnp.dot(macro, wmu, preferred_element_type=jnp.float32) + bmu
        lv = jnp.dot(macro, wlv, preferred_element_type=jnp.float32) + blv
        std = jnp.exp(0.5 * lv)
        z = mu + eps * std
        p = jnp.dot(z, wp1, preferred_element_type=jnp.float32) + bp1
        p = jnp.maximum(p, 0.0)
        py = jnp.dot(p, wp2, preferred_element_type=jnp.float32) + bp2
        mu_ref[...] = mu
        lv_ref[...] = lv
        py_ref[...] = py


@functools.partial(jax.jit, static_argnames=())
def kernel(nodes, edges, senders, receivers,
           W_enc1, b_enc1, W_enc2, b_enc2,
           W_dec1, b_dec1, W_dec2, b_dec2,
           W_mu, b_mu, W_lv, b_lv,
           W_p1, b_p1, W_p2, b_p2):
    eps = jax.random.normal(jax.random.PRNGKey(0), (LATENT,), jnp.float32)

    padr = lambda w: jnp.pad(w, ((0, 0), (0, 128 - w.shape[1])))
    vec = lambda v: jnp.pad(v, (0, 128 - v.shape[0]))
    vecs = jnp.stack([vec(b_enc1), vec(b_enc2), vec(b_dec1), vec(b_dec2),
                      vec(b_mu), vec(b_lv), vec(b_p1), vec(b_p2), vec(eps),
                      jnp.zeros(128, jnp.float32),
                      jnp.zeros(128, jnp.float32),
                      jnp.zeros(128, jnp.float32),
                      jnp.zeros(128, jnp.float32),
                      jnp.zeros(128, jnp.float32),
                      jnp.zeros(128, jnp.float32),
                      jnp.zeros(128, jnp.float32)], axis=0)
    packed = jnp.concatenate(
        [padr(W_dec1), padr(W_dec2), padr(W_mu), padr(W_lv),
         padr(W_p1), padr(W_p2), vecs], axis=0)

    full = lambda shape: pl.BlockSpec(shape, lambda i: (0, 0))
    grid = N // BLK

    out = pl.pallas_call(
        _dense_kernel,
        grid=(grid,),
        in_specs=[pl.BlockSpec((BLK, D), lambda i: (i, 0)),
                  full((D, D)), full((D, D)), full((_R_TOT, 128))],
        out_specs=[
            pl.BlockSpec((BLK, CLUSTERS), lambda i: (i, 0)),
            pl.BlockSpec((CLUSTERS, H2), lambda i: (0, 0)),
            full((1, LATENT)),
            full((1, LATENT)),
            full((1, 1)),
        ],
        out_shape=[
            jax.ShapeDtypeStruct((N, CLUSTERS), jnp.float32),
            jax.ShapeDtypeStruct((CLUSTERS, H2), jnp.float32),
            jax.ShapeDtypeStruct((1, LATENT), jnp.float32),
            jax.ShapeDtypeStruct((1, LATENT), jnp.float32),
            jax.ShapeDtypeStruct((1, 1), jnp.float32),
        ],
        compiler_params=pltpu.CompilerParams(
            dimension_semantics=("arbitrary",),
        ),
    )(nodes, W_enc1, W_enc2, packed)

    assignments, coarse_nodes, mu, lv, py = out
    return (mu.reshape(LATENT), lv.reshape(LATENT), py.reshape(1),
            assignments, coarse_nodes)


# speedup vs baseline: 1.3098x; 1.3098x over previous
import functools

import numpy as np
import jax
import jax.numpy as jnp
from jax.experimental import pallas as pl
from jax.experimental.pallas import tpu as pltpu

N = 10000
D = 128
H2 = 128
CLUSTERS = 64
LATENT = 64
BLK = 2000

_EPS = np.asarray(jax.random.normal(jax.random.PRNGKey(0), (LATENT,), jnp.float32))


def _dense_kernel(nodes_ref, w1_ref, w2_ref, wd2_ref, wml_ref, wdp_ref,
                  wp2_ref, b1_ref, b2_ref, bd1_ref, bd2_ref, bmu_ref,
                  blv_ref, bp1_ref, bp2_ref, eps_ref,
                  assign_ref, coarse_ref, mu_ref, lv_ref, py_ref):
    i = pl.program_id(0)
    x = nodes_ref[...]
    h = jnp.dot(x, w1_ref[...], preferred_element_type=jnp.float32) + b1_ref[...]
    h = jnp.maximum(h, 0.0)
    h = jnp.dot(h, w2_ref[...], preferred_element_type=jnp.float32) + b2_ref[...]

    wd1 = wdp_ref[:128, :]
    a = jnp.dot(h, wd1, preferred_element_type=jnp.float32) + bd1_ref[...]
    a = jnp.maximum(a, 0.0)
    logits = jnp.dot(a, wd2_ref[...], preferred_element_type=jnp.float32) + bd2_ref[...]
    m = jnp.max(logits, axis=-1, keepdims=True)
    e = jnp.exp(logits - m)
    assign = e / jnp.sum(e, axis=-1, keepdims=True)
    assign_ref[...] = assign

    partial = jax.lax.dot_general(assign, h, (((0,), (0,)), ((), ())),
                                  preferred_element_type=jnp.float32)

    @pl.when(i == 0)
    def _():
        coarse_ref[...] = partial

    @pl.when(i > 0)
    def _():
        coarse_ref[...] += partial

    @pl.when(i == pl.num_programs(0) - 1)
    def _():
        coarse = coarse_ref[...]
        macro = jnp.mean(coarse, axis=0, keepdims=True)
        wmu = wml_ref[:128, :]
        wlv = wml_ref[128:, :]
        wp1 = wdp_ref[128:, :]
        mu = jnp.dot(macro, wmu, preferred_element_type=jnp.float32) + bmu_ref[...]
        lv = jnp.dot(macro, wlv, preferred_element_type=jnp.float32) + blv_ref[...]
        std = jnp.exp(0.5 * lv)
        z = mu + eps_ref[...] * std
        p = jnp.dot(z, wp1, preferred_element_type=jnp.float32) + bp1_ref[...]
        p = jnp.maximum(p, 0.0)
        py = jnp.dot(p, wp2_ref[...], preferred_element_type=jnp.float32) + bp2_ref[...]
        mu_ref[...] = mu
        lv_ref[...] = lv
        py_ref[...] = py


@functools.partial(jax.jit, static_argnames=())
def kernel(nodes, edges, senders, receivers,
           W_enc1, b_enc1, W_enc2, b_enc2,
           W_dec1, b_dec1, W_dec2, b_dec2,
           W_mu, b_mu, W_lv, b_lv,
           W_p1, b_p1, W_p2, b_p2):
    w_ml = jnp.concatenate([W_mu, W_lv], axis=0)
    w_dp = jnp.concatenate([W_dec1, W_p1], axis=0)
    eps = jnp.asarray(_EPS).reshape(1, LATENT)

    row = lambda v: v.reshape(1, -1)
    full = lambda arr: pl.BlockSpec(arr.shape, lambda i: (0, 0))
    grid = N // BLK

    consts = (W_enc1, W_enc2, W_dec2, w_ml, w_dp, W_p2,
              row(b_enc1), row(b_enc2), row(b_dec1), row(b_dec2),
              row(b_mu), row(b_lv), row(b_p1), row(b_p2), eps)

    out = pl.pallas_call(
        _dense_kernel,
        grid=(grid,),
        in_specs=[pl.BlockSpec((BLK, D), lambda i: (i, 0))] + [full(c) for c in consts],
        out_specs=[
            pl.BlockSpec((BLK, CLUSTERS), lambda i: (i, 0)),
            pl.BlockSpec((CLUSTERS, H2), lambda i: (0, 0)),
            pl.BlockSpec((1, LATENT), lambda i: (0, 0)),
            pl.BlockSpec((1, LATENT), lambda i: (0, 0)),
            pl.BlockSpec((1, 1), lambda i: (0, 0)),
        ],
        out_shape=[
            jax.ShapeDtypeStruct((N, CLUSTERS), jnp.float32),
            jax.ShapeDtypeStruct((CLUSTERS, H2), jnp.float32),
            jax.ShapeDtypeStruct((1, LATENT), jnp.float32),
            jax.ShapeDtypeStruct((1, LATENT), jnp.float32),
            jax.ShapeDtypeStruct((1, 1), jnp.float32),
        ],
        compiler_params=pltpu.CompilerParams(
            dimension_semantics=("arbitrary",),
        ),
    )(nodes, *consts)

    assignments, coarse_nodes, mu, lv, py = out
    return (mu.reshape(LATENT), lv.reshape(LATENT), py.reshape(1),
            assignments, coarse_nodes)
